# Initial kernel scaffold; baseline (speedup 1.0000x reference)
import jax
import jax.numpy as jnp
from jax.experimental import pallas as pl


def kernel(prev, x, edge_index, W, att_src, att_dst, bias, gamma, beta):
    raise NotImplementedError("write your pallas kernel here")


# speedup vs baseline: 67.0667x; 67.0667x over previous
import functools

import jax
import jax.numpy as jnp
import numpy as np
from jax import lax
from jax.experimental import pallas as pl
from jax.experimental.pallas import tpu as pltpu
from jax.experimental.pallas import tpu_sc as plsc

N = 10000
E = 320000
F = 128
H = 8
C = 16

NC = 2
NS = 16
NW = NC * NS
EPT = E // NW
K = 80
NCH = EPT // K
NP = 10240
RPT = NP // NS
RB = 128

_G_NP = np.kron(np.eye(H), np.ones((C, 1))).astype(np.float32)
_R_NP = np.eye(H)[::-1].astype(np.float32).copy()


def _leaky(v):
    return jnp.where(v >= 0, v, v * jnp.float32(0.2))


def _pre_body(x_ref, w_ref, asf_ref, adf_ref, g_ref, r_ref,
              h_ref, a_ref, b_ref, es_ref):
    h = jnp.dot(x_ref[...], w_ref[...], preferred_element_type=jnp.float32)
    h_ref[...] = h
    g = g_ref[...]
    asrc = jnp.dot(h * asf_ref[...], g, preferred_element_type=jnp.float32)
    adst = jnp.dot(h * adf_ref[...], g, preferred_element_type=jnp.float32)
    gmax = jnp.max(asrc, axis=0, keepdims=True)
    shift = _leaky(gmax + adst)
    shrev = jnp.dot(shift, r_ref[...], preferred_element_type=jnp.float32)
    a_ref[...] = jnp.concatenate([asrc, asrc], axis=1)
    b_ref[...] = jnp.concatenate([adst, shrev], axis=1)
    es_ref[...] = jnp.exp(_leaky(asrc + adst) - shift)


_pre = pl.pallas_call(
    _pre_body,
    out_shape=[
        jax.ShapeDtypeStruct((N, F), jnp.float32),
        jax.ShapeDtypeStruct((N, 2 * H), jnp.float32),
        jax.ShapeDtypeStruct((N, 2 * H), jnp.float32),
        jax.ShapeDtypeStruct((N, H), jnp.float32),
    ],
)


_mesh = plsc.VectorSubcoreMesh(core_axis_name="c", subcore_axis_name="s")


@functools.partial(
    pl.kernel,
    out_type=[
        jax.ShapeDtypeStruct((NC, NP, F), jnp.float32),
        jax.ShapeDtypeStruct((NC, NP, C), jnp.float32),
    ],
    mesh=_mesh,
    compiler_params=pltpu.CompilerParams(use_tc_tiling_on_sc=False),
    scratch_types=[
        pltpu.VMEM((K,), jnp.int32),
        pltpu.VMEM((K,), jnp.int32),
        pltpu.VMEM((K, 16), jnp.float32),
        pltpu.VMEM((K, 16), jnp.float32),
        pltpu.VMEM((K, F), jnp.float32),
        pltpu.VMEM((K, 16), jnp.float32),
        pltpu.VMEM((RB, F), jnp.float32),
        pltpu.VMEM((RB, 16), jnp.float32),
        pltpu.VMEM_SHARED((NP, F), jnp.float32),
        pltpu.VMEM_SHARED((NP, C), jnp.float32),
        pltpu.SemaphoreType.DMA,
        pltpu.SemaphoreType.DMA,
        pltpu.SemaphoreType.DMA,
    ],
)
def _sweep(src_hbm, dst_hbm, h_hbm, a_hbm, b_hbm,
           num_hbm, den_hbm,
           sidx, didx, abuf, bbuf, hbuf, exbuf, stg, stg16,
           num_sh, den_sh, sem_a, sem_b, sem_h):
    c = lax.axis_index("c")
    s = lax.axis_index("s")
    wid = s * NC + c

    z16 = jnp.zeros((16,), jnp.float32)
    lmask = lax.iota(jnp.int32, 16) < H

    def _zero_row(r, carry):
        for j in range(F // 16):
            stg[r, pl.ds(16 * j, 16)] = z16
        stg16[r, :] = z16
        return carry

    lax.fori_loop(0, RB, _zero_row, 0)
    for b in range(RPT // RB):
        r0 = s * RPT + b * RB
        pltpu.sync_copy(stg, num_sh.at[pl.ds(r0, RB)])
        pltpu.sync_copy(stg16, den_sh.at[pl.ds(r0, RB)])
    plsc.subcore_barrier()

    def _chunk(ci, carry):
        pltpu.sync_copy(src_hbm.at[wid, ci], sidx)
        pltpu.sync_copy(dst_hbm.at[wid, ci], didx)
        ca = pltpu.async_copy(a_hbm.at[sidx], abuf, sem_a)
        cb = pltpu.async_copy(b_hbm.at[didx], bbuf, sem_b)
        ch = pltpu.async_copy(h_hbm.at[sidx], hbuf, sem_h)
        ca.wait()
        cb.wait()
        ch.wait()

        def _edge(k, carry2):
            av = abuf[k, :]
            bv = bbuf[k, :]
            sv = av + bv
            lr = jnp.where(sv >= 0, sv, sv * jnp.float32(0.2))
            shv = jnp.flip(bv, 0)
            exv = jnp.exp(lr - shv)
            exm = jnp.where(lmask, exv, jnp.float32(0.0))
            exbuf[k, :] = exm
            for j in range(H):
                hv = hbuf[k, pl.ds(C * j, 16)]
                hbuf[k, pl.ds(C * j, 16)] = hv * exm[j]
            return carry2

        lax.fori_loop(0, K, _edge, 0)
        pltpu.sync_copy(exbuf, den_sh.at[didx], add=True)
        pltpu.sync_copy(hbuf, num_sh.at[didx], add=True)
        return carry

    lax.fori_loop(0, NCH, _chunk, 0)
    plsc.subcore_barrier()

    for b in range(RPT // RB):
        r0 = s * RPT + b * RB
        pltpu.sync_copy(num_sh.at[pl.ds(r0, RB)], stg)
        pltpu.sync_copy(stg, num_hbm.at[c, pl.ds(r0, RB)])
        pltpu.sync_copy(den_sh.at[pl.ds(r0, RB)], stg16)
        pltpu.sync_copy(stg16, den_hbm.at[c, pl.ds(r0, RB)])


def _post_body(prev_ref, num_ref, den_ref, es_ref, h_ref, gt_ref,
               bias_ref, gamma_ref, beta_ref, out_ref):
    es = es_ref[...]
    gt = gt_ref[...]
    den8 = den_ref[0, :N, :H] + den_ref[1, :N, :H] + es
    den128 = jnp.dot(den8, gt, preferred_element_type=jnp.float32)
    exp128 = jnp.dot(es, gt, preferred_element_type=jnp.float32)
    numtot = num_ref[0, :N, :] + num_ref[1, :N, :] + exp128 * h_ref[...]
    y = prev_ref[...] + numtot / (den128 + jnp.float32(1e-16)) + bias_ref[...]
    mean = jnp.mean(y, axis=0, keepdims=True)
    var = jnp.mean(y * y, axis=0, keepdims=True) - mean * mean
    yn = (y - mean) * lax.rsqrt(var + jnp.float32(1e-5))
    out_ref[...] = jnp.maximum(yn * gamma_ref[...] + beta_ref[...],
                               jnp.float32(0.0))


_post = pl.pallas_call(
    _post_body,
    out_shape=jax.ShapeDtypeStruct((N, F), jnp.float32),
)


def kernel(prev, x, edge_index, W, att_src, att_dst, bias, gamma, beta):
    src3 = edge_index[0].reshape(NW, NCH, K)
    dst3 = edge_index[1].reshape(NW, NCH, K)
    g = jnp.asarray(_G_NP)
    r = jnp.asarray(_R_NP)
    h, A, B, es = _pre(x, W, att_src.reshape(1, F), att_dst.reshape(1, F),
                       g, r)
    num, den = _sweep(src3, dst3, h, A, B)
    return _post(prev, num, den, es, h, g.T,
                 bias.reshape(1, F), gamma.reshape(1, F), beta.reshape(1, F))
